# baseline (speedup 1.0000x reference)
import functools

import jax
import jax.numpy as jnp
from jax import lax
from jax.experimental import pallas as pl
from jax.experimental.pallas import tpu as pltpu
from jax.experimental.pallas import tpu_sc as plsc

_N = 20000
_B = 512
_NW = 32
_CH = 160
_NCHUNK = _N // _CH
_KMAX = (_NCHUNK + _NW - 1) // _NW


def _lane_gather(x, idx):
    dn = lax.GatherDimensionNumbers(
        offset_dims=(), collapsed_slice_dims=(0,), start_index_map=(0,))
    return lax.gather(x, idx[:, None], dn, (1,),
                      mode=lax.GatherScatterMode.PROMISE_IN_BOUNDS)


def _sc_body(b1_hbm, b2_hbm, ious_hbm, amax_hbm, b1v, b2v, rowbuf, amaxbuf):
    wid = lax.axis_index("s") * 2 + lax.axis_index("c")
    pltpu.sync_copy(b2_hbm, b2v)
    lane = lax.iota(jnp.int32, 16)

    def row_step(i, acc):
        row = b1v[i, :]
        im_a = row[0]
        x1a = row[1]
        y1a = row[2]
        x2a = row[3]
        y2a = row[4]
        area_a = row[5]
        best_v = jnp.full((16,), -1.0, jnp.float32)
        best_c = jnp.zeros((16,), jnp.int32)
        for j in range(_B // 16):
            sl = pl.ds(j * 16, 16)
            im_b = b2v[0, sl]
            x1b = b2v[1, sl]
            y1b = b2v[2, sl]
            x2b = b2v[3, sl]
            y2b = b2v[4, sl]
            area_b = b2v[5, sl]
            iw = jnp.maximum(jnp.minimum(x2a, x2b) - jnp.maximum(x1a, x1b), 0.0)
            ih = jnp.maximum(jnp.minimum(y2a, y2b) - jnp.maximum(y1a, y1b), 0.0)
            inter = iw * ih
            iou = inter / ((area_a + area_b) - inter)
            iou = jnp.where(im_a != im_b, 0.0, iou)
            rowbuf[i, sl] = iou
            upd = iou > best_v
            best_v = jnp.where(upd, iou, best_v)
            best_c = jnp.where(upd, j * 16 + lane, best_c)
        for s in (8, 4, 2, 1):
            idx = jnp.bitwise_xor(lane, s)
            ov = _lane_gather(best_v, idx)
            oc = _lane_gather(best_c, idx)
            take = (ov > best_v) | ((ov == best_v) & (oc < best_c))
            best_v = jnp.where(take, ov, best_v)
            best_c = jnp.where(take, oc, best_c)
        acc = jnp.where(lane == (i % 16), best_c, acc)

        @pl.when(i % 16 == 15)
        def _():
            amaxbuf[pl.ds((i // 16) * 16, 16)] = acc

        return acc

    for k in range(_KMAX):
        c = wid + k * _NW

        @pl.when(c < _NCHUNK)
        def _():
            base = c * _CH
            pltpu.sync_copy(b1_hbm.at[pl.ds(base, _CH), :], b1v)
            lax.fori_loop(0, _CH, row_step, jnp.zeros((16,), jnp.int32))
            pltpu.sync_copy(rowbuf, ious_hbm.at[pl.ds(base, _CH), :])
            pltpu.sync_copy(amaxbuf, amax_hbm.at[pl.ds(base, _CH)])


def _pack(boxes, width=6):
    im = boxes[:, 0:1]
    x1 = boxes[:, 1:2]
    y1 = boxes[:, 2:3]
    x2 = boxes[:, 3:4]
    y2 = boxes[:, 4:5]
    area = (x2 - x1 + 1.0) * (y2 - y1 + 1.0)
    cols = [im, x1, y1, x2 + 1.0, y2 + 1.0, area]
    if width > 6:
        cols.append(jnp.zeros((boxes.shape[0], width - 6), jnp.float32))
    return jnp.concatenate(cols, axis=1)


@functools.partial(jax.jit, static_argnames=())
def kernel(boxes1, boxes2):
    b1p = _pack(boxes1, 16)
    b2p = _pack(boxes2).T
    sc = functools.partial(
        pl.kernel,
        out_type=[
            jax.ShapeDtypeStruct((_N, _B), jnp.float32),
            jax.ShapeDtypeStruct((_N,), jnp.int32),
        ],
        mesh=plsc.VectorSubcoreMesh(core_axis_name="c", subcore_axis_name="s"),
        scratch_types=[
            pltpu.VMEM((_CH, 16), jnp.float32),
            pltpu.VMEM((6, _B), jnp.float32),
            pltpu.VMEM((_CH, _B), jnp.float32),
            pltpu.VMEM((_CH,), jnp.int32),
        ],
    )(_sc_body)
    ious, amax = sc(b1p, b2p)
    return amax, ious


# speedup vs baseline: 6.3187x; 6.3187x over previous
import functools

import jax
import jax.numpy as jnp
from jax.experimental import pallas as pl
from jax.experimental.pallas import tpu as pltpu

_N = 20000
_B = 512
_ROWS = 2000


def _iou_kernel(b1_ref, b2t_ref, ious_ref, amax_ref):
    b1 = b1_ref[...]
    b2 = b2t_ref[...]

    im_a = b1[:, 0:1]
    x1a = b1[:, 1:2]
    y1a = b1[:, 2:3]
    x2a = b1[:, 3:4]
    y2a = b1[:, 4:5]
    area_a = b1[:, 5:6]

    im_b = b2[0:1, :]
    x1b = b2[1:2, :]
    y1b = b2[2:3, :]
    x2b = b2[3:4, :]
    y2b = b2[4:5, :]
    area_b = b2[5:6, :]

    iw = jnp.maximum(jnp.minimum(x2a, x2b) - jnp.maximum(x1a, x1b), 0.0)
    ih = jnp.maximum(jnp.minimum(y2a, y2b) - jnp.maximum(y1a, y1b), 0.0)
    inter = iw * ih
    iou = inter / ((area_a + area_b) - inter)
    iou = jnp.where(im_a != im_b, 0.0, iou)
    ious_ref[...] = iou

    v0, v1, v2, v3 = (iou[:, i * 128:(i + 1) * 128] for i in range(4))
    m01 = jnp.maximum(v0, v1)
    g01 = (v1 > v0).astype(jnp.int32)
    m23 = jnp.maximum(v2, v3)
    g23 = jnp.where(v3 > v2, 3, 2)
    m = jnp.maximum(m01, m23)
    gg = jnp.where(m23 > m01, g23, g01)
    lane = jax.lax.broadcasted_iota(jnp.int32, m.shape, 1)
    colc = gg * 128 + lane
    mx = jnp.max(m, axis=1, keepdims=True)
    amax_ref[...] = jnp.min(
        jnp.where(m == mx, colc, _B), axis=1, keepdims=True
    )


def _pack(boxes):
    im = boxes[:, 0:1]
    x1 = boxes[:, 1:2]
    y1 = boxes[:, 2:3]
    x2 = boxes[:, 3:4]
    y2 = boxes[:, 4:5]
    area = (x2 - x1 + 1.0) * (y2 - y1 + 1.0)
    return jnp.concatenate([im, x1, y1, x2 + 1.0, y2 + 1.0, area], axis=1)


@functools.partial(jax.jit, static_argnames=())
def kernel(boxes1, boxes2):
    b1p = _pack(boxes1)
    b2p = _pack(boxes2).T
    grid = (pl.cdiv(_N, _ROWS),)
    ious, amax = pl.pallas_call(
        _iou_kernel,
        grid=grid,
        in_specs=[
            pl.BlockSpec((_ROWS, 6), lambda i: (i, 0)),
            pl.BlockSpec((6, _B), lambda i: (0, 0)),
        ],
        out_specs=[
            pl.BlockSpec((_ROWS, _B), lambda i: (i, 0)),
            pl.BlockSpec((_ROWS, 1), lambda i: (i, 0)),
        ],
        out_shape=[
            jax.ShapeDtypeStruct((_N, _B), jnp.float32),
            jax.ShapeDtypeStruct((_N, 1), jnp.int32),
        ],
        compiler_params=pltpu.CompilerParams(
            dimension_semantics=("parallel",),
        ),
    )(b1p, b2p)
    return amax.reshape(_N), ious
